# baseline (speedup 1.0000x reference)
import functools
import jax
import jax.numpy as jnp
from jax import lax
from jax.experimental import pallas as pl
from jax.experimental.pallas import tpu as pltpu
from jax.experimental.pallas import tpu_sc as plsc

BATCH = 4096
HIST = 50
D = 128
NW = 32
NS = 16
COLS = BATCH // NW
NBUF = 3
LOOKAHEAD = 2

_mesh = plsc.VectorSubcoreMesh(core_axis_name="c", subcore_axis_name="s")


@functools.partial(
    pl.kernel,
    mesh=_mesh,
    out_type=jax.ShapeDtypeStruct((HIST, BATCH, D), jnp.float32),
    scratch_types=[
        pltpu.VMEM((HIST, COLS), jnp.int32),
        [pltpu.VMEM((COLS, D), jnp.float32) for _ in range(NBUF)],
        [pltpu.VMEM_SHARED((NS, COLS, D), jnp.float32) for _ in range(NBUF)],
        [pltpu.SemaphoreType.DMA for _ in range(NBUF)],
        [pltpu.SemaphoreType.DMA for _ in range(NBUF)],
        [pltpu.SemaphoreType.DMA for _ in range(NBUF)],
    ],
)
def _gather_kernel(idx_hbm, table_hbm, out_hbm, idx_v, tbufs, sbufs,
                   gsems, csems, wsems):
    sid = lax.axis_index("s")
    wid = sid * 2 + lax.axis_index("c")
    col0 = wid * COLS
    pltpu.sync_copy(idx_hbm.at[:, wid], idx_v)

    def gather(l, b):
        pltpu.async_copy(table_hbm.at[idx_v.at[l]], tbufs[b], gsems[b])

    def gather_wait(l, b):
        pltpu.make_async_copy(table_hbm.at[idx_v.at[l]], tbufs[b], gsems[b]).wait()

    def xcopy(l, b):
        pltpu.async_copy(tbufs[b], sbufs[b].at[sid], csems[b])

    def xcopy_wait(l, b):
        pltpu.make_async_copy(tbufs[b], sbufs[b].at[sid], csems[b]).wait()

    def write(l, b):
        pltpu.async_copy(sbufs[b].at[sid], out_hbm.at[l, pl.ds(col0, COLS)],
                         wsems[b])

    def write_wait(l, b):
        pltpu.make_async_copy(sbufs[b].at[sid],
                              out_hbm.at[l, pl.ds(col0, COLS)], wsems[b]).wait()

    def iteration(l):
        if l >= NBUF:
            write_wait(l - NBUF, (l - NBUF) % NBUF)
        gather_wait(l, l % NBUF)
        xcopy(l, l % NBUF)
        if l >= 1:
            xcopy_wait(l - 1, (l - 1) % NBUF)
            write(l - 1, (l - 1) % NBUF)
        if l + LOOKAHEAD < HIST:
            gather(l + LOOKAHEAD, (l + LOOKAHEAD) % NBUF)

    for l in range(LOOKAHEAD):
        gather(l, l % NBUF)

    for l in range(NBUF):
        iteration(l)

    MAIN_END = NBUF + ((HIST - LOOKAHEAD - NBUF) // NBUF) * NBUF

    @pl.loop(NBUF, MAIN_END, step=NBUF)
    def _main(l0):
        for d in range(NBUF):
            l = l0 + d
            write_wait(l - NBUF, d)
            gather_wait(l, d)
            xcopy(l, d)
            bp = (d - 1) % NBUF
            xcopy_wait(l - 1, bp)
            write(l - 1, bp)
            gather(l + LOOKAHEAD, (d + LOOKAHEAD) % NBUF)

    for l in range(NBUF + ((HIST - LOOKAHEAD - NBUF) // NBUF) * NBUF, HIST):
        iteration(l)

    xcopy_wait(HIST - 1, (HIST - 1) % NBUF)
    write(HIST - 1, (HIST - 1) % NBUF)
    for l in range(HIST - NBUF, HIST):
        write_wait(l, l % NBUF)


def kernel(x, target, emb_table):
    idx = jnp.transpose(target.astype(jnp.int32)).reshape(HIST, NW, COLS)
    out_t = _gather_kernel(idx, emb_table)
    return jnp.transpose(out_t, (1, 0, 2))
